# baseline (speedup 1.0000x reference)
import functools

import jax
import jax.numpy as jnp
from jax import lax
from jax.experimental import pallas as pl
from jax.experimental.pallas import tpu as pltpu
from jax.experimental.pallas import tpu_sc as plsc

_NC = 2
_NS = 16
_L = 16
_CH = 32
_NB = 2


def _embed_stem(idx_flat, tok_emb, pos):
    BT = idx_flat.shape[0]
    T, D = pos.shape
    NW = _NC * _NS
    RPW = BT // NW
    NCH = RPW // _CH

    mesh = plsc.VectorSubcoreMesh(core_axis_name="c", subcore_axis_name="s")

    @functools.partial(
        pl.kernel,
        mesh=mesh,
        out_type=jax.ShapeDtypeStruct((BT, D), jnp.float32),
        scratch_types=[
            pltpu.VMEM((RPW,), jnp.int32),
            pltpu.VMEM((_NB, _CH, D), jnp.float32),
            pltpu.VMEM((_NB, _CH, D), jnp.float32),
            pltpu.SemaphoreType.DMA((_NB,)),
            pltpu.SemaphoreType.DMA((_NB,)),
            pltpu.SemaphoreType.DMA((_NB,)),
        ],
    )
    def k(idx_hbm, tab_hbm, pos_hbm, out_hbm, idx_v, gbuf, pbuf,
          gsem, psem, osem):
        wid = lax.axis_index("s") * _NC + lax.axis_index("c")
        base = wid * RPW
        t0 = base % T
        pltpu.sync_copy(idx_hbm.at[pl.ds(base, RPW)], idx_v)

        def start_in(kk):
            bb = kk % _NB
            g = pltpu.async_copy(
                tab_hbm.at[idx_v.at[pl.ds(kk * _CH, _CH)]],
                gbuf.at[bb], gsem.at[bb])
            p = pltpu.async_copy(
                pos_hbm.at[pl.ds(t0 + kk * _CH, _CH)],
                pbuf.at[bb], psem.at[bb])
            return g, p

        inflight = {0: start_in(0)}
        outs = {}
        for kk in range(NCH):
            bb = kk % _NB
            g, p = inflight.pop(kk)
            g.wait()
            p.wait()
            if kk + 1 < NCH:
                if kk >= 1:
                    outs.pop(kk - 1).wait()
                inflight[kk + 1] = start_in(kk + 1)

            def row_add(r, carry):
                for c in range(D // _L):
                    sl = pl.ds(c * _L, _L)
                    gbuf[bb, r, sl] = gbuf[bb, r, sl] + pbuf[bb, r, sl]
                return carry

            lax.fori_loop(0, _CH, row_add, 0)
            outs[kk] = pltpu.async_copy(
                gbuf.at[bb], out_hbm.at[pl.ds(base + kk * _CH, _CH)],
                osem.at[bb])
        outs.pop(NCH - 2).wait()
        outs.pop(NCH - 1).wait()

    return k(idx_flat, tok_emb, pos)


def kernel(idx, tok_emb, pos_embed):
    b, t = idx.shape
    d = tok_emb.shape[1]
    pos = pos_embed[0, :t, :]
    out = _embed_stem(idx.reshape(-1).astype(jnp.int32), tok_emb, pos)
    return out.reshape(b, t, d)


# speedup vs baseline: 1.1083x; 1.1083x over previous
import functools

import jax
import jax.numpy as jnp
from jax import lax
from jax.experimental import pallas as pl
from jax.experimental.pallas import tpu as pltpu
from jax.experimental.pallas import tpu_sc as plsc

_NC = 2
_NS = 16
_L = 16
_CH = 64


def _embed_stem(idx_flat, tok_emb, pos):
    BT = idx_flat.shape[0]
    T, D = pos.shape
    NW = _NC * _NS
    RPW = BT // NW
    NCH = RPW // _CH

    mesh = plsc.VectorSubcoreMesh(core_axis_name="c", subcore_axis_name="s")

    @functools.partial(
        pl.kernel,
        mesh=mesh,
        out_type=jax.ShapeDtypeStruct((BT, D), jnp.float32),
        scratch_types=[
            pltpu.VMEM((RPW,), jnp.int32),
            pltpu.VMEM((_CH, D), jnp.float32),
            pltpu.VMEM((_CH, D), jnp.float32),
            pltpu.SemaphoreType.DMA,
            pltpu.SemaphoreType.DMA,
        ],
    )
    def k(idx_hbm, tab_hbm, pos_hbm, out_hbm, idx_v, gbuf, pbuf, gsem, psem):
        wid = lax.axis_index("s") * _NC + lax.axis_index("c")
        base = wid * RPW
        t0 = base % T
        pltpu.sync_copy(idx_hbm.at[pl.ds(base, RPW)], idx_v)
        for kk in range(NCH):
            g = pltpu.async_copy(
                tab_hbm.at[idx_v.at[pl.ds(kk * _CH, _CH)]], gbuf, gsem)
            p = pltpu.async_copy(
                pos_hbm.at[pl.ds(t0 + kk * _CH, _CH)], pbuf, psem)
            g.wait()
            p.wait()

            @plsc.parallel_loop(0, _CH, unroll=2)
            def row_add(r):
                for c in range(D // _L):
                    sl = pl.ds(c * _L, _L)
                    gbuf[r, sl] = gbuf[r, sl] + pbuf[r, sl]

            pltpu.sync_copy(gbuf, out_hbm.at[pl.ds(base + kk * _CH, _CH)])

    return k(idx_flat, tok_emb, pos)


def kernel(idx, tok_emb, pos_embed):
    b, t = idx.shape
    d = tok_emb.shape[1]
    pos = pos_embed[0, :t, :]
    out = _embed_stem(idx.reshape(-1).astype(jnp.int32), tok_emb, pos)
    return out.reshape(b, t, d)


# speedup vs baseline: 1.2834x; 1.1580x over previous
import functools

import jax
import jax.numpy as jnp
from jax import lax
from jax.experimental import pallas as pl
from jax.experimental.pallas import tpu as pltpu
from jax.experimental.pallas import tpu_sc as plsc

_NC = 2
_NS = 16
_L = 16
_CH = 32
_NB = 3


def _embed_stem(idx_flat, tok_emb, pos):
    BT = idx_flat.shape[0]
    T, D = pos.shape
    B = BT // T
    NW = _NC * _NS
    TPW = T // NW
    NCT = TPW // _CH
    NCH = B * NCT

    mesh = plsc.VectorSubcoreMesh(core_axis_name="c", subcore_axis_name="s")

    @functools.partial(
        pl.kernel,
        mesh=mesh,
        out_type=jax.ShapeDtypeStruct((BT, D), jnp.float32),
        scratch_types=[
            pltpu.VMEM((B * TPW,), jnp.int32),
            pltpu.VMEM((_NB, _CH, D), jnp.float32),
            pltpu.VMEM((TPW, D), jnp.float32),
            pltpu.SemaphoreType.DMA((_NB,)),
            pltpu.SemaphoreType.DMA((_NB,)),
            pltpu.SemaphoreType.DMA,
        ],
    )
    def k(idx_hbm, tab_hbm, pos_hbm, out_hbm, idx_v, gbuf, pbuf,
          gsem, osem, psem):
        wid = lax.axis_index("s") * _NC + lax.axis_index("c")
        t0 = wid * TPW
        pos_cp = pltpu.async_copy(pos_hbm.at[pl.ds(t0, TPW)], pbuf, psem)
        for b4 in range(B):
            pltpu.sync_copy(idx_hbm.at[pl.ds(b4 * T + t0, TPW)],
                            idx_v.at[pl.ds(b4 * TPW, TPW)])

        def row_of(kk):
            b4, tt = kk // NCT, kk % NCT
            return b4 * T + t0 + tt * _CH

        def start_gather(kk):
            bb = kk % _NB
            return pltpu.async_copy(
                tab_hbm.at[idx_v.at[pl.ds(kk * _CH, _CH)]],
                gbuf.at[bb], gsem.at[bb])

        gathers = {0: start_gather(0), 1: start_gather(1)}
        outs = {}
        for kk in range(NCH):
            bb = kk % _NB
            if kk + 2 < NCH:
                if kk >= 1:
                    outs.pop(kk - 1).wait()
                gathers[kk + 2] = start_gather(kk + 2)
            gathers.pop(kk).wait()
            if kk == 0:
                pos_cp.wait()
            toff = (kk % NCT) * _CH

            @plsc.parallel_loop(0, _CH, unroll=2)
            def row_add(r):
                for c in range(D // _L):
                    sl = pl.ds(c * _L, _L)
                    gbuf[bb, r, sl] = gbuf[bb, r, sl] + pbuf[toff + r, sl]

            outs[kk] = pltpu.async_copy(
                gbuf.at[bb], out_hbm.at[pl.ds(row_of(kk), _CH)], osem.at[bb])
        for kk in sorted(outs):
            outs[kk].wait()

    return k(idx_flat, tok_emb, pos)


def kernel(idx, tok_emb, pos_embed):
    b, t = idx.shape
    d = tok_emb.shape[1]
    pos = pos_embed[0, :t, :]
    out = _embed_stem(idx.reshape(-1).astype(jnp.int32), tok_emb, pos)
    return out.reshape(b, t, d)


# speedup vs baseline: 1.3352x; 1.0403x over previous
import functools

import jax
import jax.numpy as jnp
from jax import lax
from jax.experimental import pallas as pl
from jax.experimental.pallas import tpu as pltpu
from jax.experimental.pallas import tpu_sc as plsc

_NC = 2
_NS = 16
_L = 16
_CH = 32


def _embed_stem(idx_flat, tok_emb, pos):
    BT = idx_flat.shape[0]
    T, D = pos.shape
    B = BT // T
    NW = _NC * _NS
    TPW = T // NW
    NCT = TPW // _CH
    NCH = B * NCT

    mesh = plsc.VectorSubcoreMesh(core_axis_name="c", subcore_axis_name="s")

    @functools.partial(
        pl.kernel,
        mesh=mesh,
        out_type=jax.ShapeDtypeStruct((BT, D), jnp.float32),
        scratch_types=[
            pltpu.VMEM((B * TPW,), jnp.int32),
            pltpu.VMEM((2, _CH, D), jnp.float32),
            pltpu.VMEM((2, _CH, D), jnp.float32),
            pltpu.VMEM((_CH, D), jnp.float32),
            pltpu.SemaphoreType.DMA((2,)),
            pltpu.SemaphoreType.DMA((2,)),
            pltpu.SemaphoreType.DMA,
        ],
    )
    def k(idx_hbm, tab_hbm, pos_hbm, out_hbm, idx_v, gbuf, sbuf, pbuf,
          gsem, osem, psem):
        wid = lax.axis_index("s") * _NC + lax.axis_index("c")
        t0 = wid * TPW
        pos_cp = pltpu.async_copy(pos_hbm.at[pl.ds(t0, _CH)], pbuf, psem)
        for b4 in range(B):
            pltpu.sync_copy(idx_hbm.at[pl.ds(b4 * T + t0, TPW)],
                            idx_v.at[pl.ds(b4 * TPW, TPW)])

        def start_gather(kk):
            tt, b4 = kk // B, kk % B
            return pltpu.async_copy(
                tab_hbm.at[idx_v.at[pl.ds(b4 * TPW + tt * _CH, _CH)]],
                gbuf.at[kk % 2], gsem.at[kk % 2])

        gathers = {0: start_gather(0), 1: start_gather(1)}
        outs = {}
        for kk in range(NCH):
            tt, b4 = kk // B, kk % B
            bb = kk % 2
            gathers.pop(kk).wait()
            if kk == 0 or kk == B:
                pos_cp.wait()
            if kk >= 2:
                outs.pop(kk - 2).wait()

            @plsc.parallel_loop(0, _CH, unroll=2)
            def row_add(r):
                for c in range(D // _L):
                    sl = pl.ds(c * _L, _L)
                    sbuf[bb, r, sl] = gbuf[bb, r, sl] + pbuf[r, sl]

            outs[kk] = pltpu.async_copy(
                sbuf.at[bb],
                out_hbm.at[pl.ds(b4 * T + t0 + tt * _CH, _CH)],
                osem.at[bb])
            if kk + 2 < NCH:
                gathers[kk + 2] = start_gather(kk + 2)
            if kk == B - 1:
                pos_cp = pltpu.async_copy(
                    pos_hbm.at[pl.ds(t0 + _CH, _CH)], pbuf, psem)
        for kk in sorted(outs):
            outs[kk].wait()

    return k(idx_flat, tok_emb, pos)


def kernel(idx, tok_emb, pos_embed):
    b, t = idx.shape
    d = tok_emb.shape[1]
    pos = pos_embed[0, :t, :]
    out = _embed_stem(idx.reshape(-1).astype(jnp.int32), tok_emb, pos)
    return out.reshape(b, t, d)


# speedup vs baseline: 1.3498x; 1.0109x over previous
import functools

import jax
import jax.numpy as jnp
from jax import lax
from jax.experimental import pallas as pl
from jax.experimental.pallas import tpu as pltpu
from jax.experimental.pallas import tpu_sc as plsc

_NC = 2
_NS = 16
_L = 16
_CH = 32


def _embed_stem(idx_flat, tok_emb, pos):
    BT = idx_flat.shape[0]
    T, D = pos.shape
    B = BT // T
    NW = _NC * _NS
    TPW = T // NW
    NCT = TPW // _CH
    NCH = B * NCT

    mesh = plsc.VectorSubcoreMesh(core_axis_name="c", subcore_axis_name="s")

    @functools.partial(
        pl.kernel,
        mesh=mesh,
        out_type=jax.ShapeDtypeStruct((BT, D), jnp.float32),
        scratch_types=[
            pltpu.VMEM((B * TPW,), jnp.int32),
            pltpu.VMEM((4, _CH, D), jnp.float32),
            pltpu.VMEM((_CH, D), jnp.float32),
            pltpu.SemaphoreType.DMA((4,)),
            pltpu.SemaphoreType.DMA((4,)),
            pltpu.SemaphoreType.DMA,
        ],
    )
    def k(idx_hbm, tab_hbm, pos_hbm, out_hbm, idx_v, gbuf, pbuf,
          gsem, osem, psem):
        wid = lax.axis_index("s") * _NC + lax.axis_index("c")
        t0 = wid * TPW
        pos_cp = pltpu.async_copy(pos_hbm.at[pl.ds(t0, _CH)], pbuf, psem)
        for b4 in range(B):
            pltpu.sync_copy(idx_hbm.at[pl.ds(b4 * T + t0, TPW)],
                            idx_v.at[pl.ds(b4 * TPW, TPW)])

        def start_gather(kk):
            tt, b4 = kk // B, kk % B
            return pltpu.async_copy(
                tab_hbm.at[idx_v.at[pl.ds(b4 * TPW + tt * _CH, _CH)]],
                gbuf.at[kk % 4], gsem.at[kk % 4])

        gathers = {0: start_gather(0), 1: start_gather(1)}
        outs = {}
        for kk in range(NCH):
            tt, b4 = kk // B, kk % B
            bb = kk % 4
            gathers.pop(kk).wait()
            if kk == 0 or kk == B:
                pos_cp.wait()
            if kk + 2 < NCH:
                if kk >= 2:
                    outs.pop(kk - 2).wait()
                gathers[kk + 2] = start_gather(kk + 2)

            @plsc.parallel_loop(0, _CH, unroll=2)
            def row_add(r):
                for c in range(D // _L):
                    sl = pl.ds(c * _L, _L)
                    gbuf[bb, r, sl] = gbuf[bb, r, sl] + pbuf[r, sl]

            outs[kk] = pltpu.async_copy(
                gbuf.at[bb],
                out_hbm.at[pl.ds(b4 * T + t0 + tt * _CH, _CH)],
                osem.at[bb])
            if kk == B - 1:
                pos_cp = pltpu.async_copy(
                    pos_hbm.at[pl.ds(t0 + _CH, _CH)], pbuf, psem)
        for kk in sorted(outs):
            outs[kk].wait()

    return k(idx_flat, tok_emb, pos)


def kernel(idx, tok_emb, pos_embed):
    b, t = idx.shape
    d = tok_emb.shape[1]
    pos = pos_embed[0, :t, :]
    out = _embed_stem(idx.reshape(-1).astype(jnp.int32), tok_emb, pos)
    return out.reshape(b, t, d)
